# baseline (speedup 1.0000x reference)
import functools

import jax
import jax.numpy as jnp
from jax import lax
from jax.experimental import pallas as pl
from jax.experimental.pallas import tpu as pltpu
from jax.experimental.pallas import tpu_sc as plsc

_B, _D, _H, _K = 4096, 512, 512, 16
_LANES = 16


def _build_mt_sparsecore(idx_flat, w1_flat):
    info = plsc.get_sparse_core_info()
    nw = info.num_cores * info.num_subcores
    th = _H // nw
    blk = _K * th
    rowlen = th * _D

    mesh = plsc.VectorSubcoreMesh(core_axis_name="c", subcore_axis_name="s")

    @functools.partial(
        pl.kernel,
        mesh=mesh,
        compiler_params=pltpu.CompilerParams(needs_layout_passes=False),
        out_type=jax.ShapeDtypeStruct((_H * _D,), jnp.float32),
        scratch_types=[
            pltpu.VMEM((blk,), jnp.int32),
            pltpu.VMEM((blk,), jnp.float32),
            pltpu.VMEM((rowlen,), jnp.float32),
        ],
    )
    def build(idx_hbm, w_hbm, m_hbm, idx_v, w_v, m_v):
        wid = lax.axis_index("s") * info.num_cores + lax.axis_index("c")
        pltpu.sync_copy(idx_hbm.at[pl.ds(wid * blk, blk)], idx_v)
        pltpu.sync_copy(w_hbm.at[pl.ds(wid * blk, blk)], w_v)

        def zero_chunk(i, c):
            m_v[pl.ds(i * _LANES, _LANES)] = jnp.zeros((_LANES,), jnp.float32)
            return c

        lax.fori_loop(0, rowlen // _LANES, zero_chunk, 0)

        row_off = lax.broadcasted_iota(jnp.int32, (_LANES,), 0) * _D
        for k in range(_K):
            addr = row_off + idx_v[pl.ds(k * _LANES, _LANES)]
            plsc.addupdate_scatter(m_v, [addr], w_v[pl.ds(k * _LANES, _LANES)])

        pltpu.sync_copy(m_v, m_hbm.at[pl.ds(wid * rowlen, rowlen)])

    return build(idx_flat, w1_flat)


def _forward_body(x_ref, mt_ref, b1_ref, w2_ref, b2_ref, ha_ref, out_ref):
    dims = (((1,), (1,)), ((), ()))
    xf = x_ref[...]
    mf = mt_ref[...]
    xh = xf.astype(jnp.bfloat16)
    mh = mf.astype(jnp.bfloat16)
    xl = (xf - xh.astype(jnp.float32)).astype(jnp.bfloat16)
    ml = (mf - mh.astype(jnp.float32)).astype(jnp.bfloat16)
    h = lax.dot_general(
        xh, mh, dimension_numbers=dims, preferred_element_type=jnp.float32
    )
    h += lax.dot_general(
        xh, ml, dimension_numbers=dims, preferred_element_type=jnp.float32
    )
    h += lax.dot_general(
        xl, mh, dimension_numbers=dims, preferred_element_type=jnp.float32
    )
    ha = jnp.tanh(h + b1_ref[...])
    ha_ref[...] = ha
    o = jnp.sum(ha * w2_ref[...], axis=1, keepdims=True)
    out_ref[...] = jnp.tanh(o + b2_ref[...])


def _forward_tensorcore(x, mt, b1, w2, b2):
    bb = 512
    grid = (_B // bb,)
    ha, out = pl.pallas_call(
        _forward_body,
        grid=grid,
        in_specs=[
            pl.BlockSpec((bb, _D), lambda i: (i, 0)),
            pl.BlockSpec((_H, _D), lambda i: (0, 0)),
            pl.BlockSpec((1, _H), lambda i: (0, 0)),
            pl.BlockSpec((1, _H), lambda i: (0, 0)),
            pl.BlockSpec((1, 1), lambda i: (0, 0)),
        ],
        out_specs=[
            pl.BlockSpec((bb, _H), lambda i: (i, 0)),
            pl.BlockSpec((bb, 1), lambda i: (i, 0)),
        ],
        out_shape=[
            jax.ShapeDtypeStruct((_B, _H), jnp.float32),
            jax.ShapeDtypeStruct((_B, 1), jnp.float32),
        ],
    )(x, mt, b1.reshape(1, _H), w2.reshape(1, _H), b2.reshape(1, 1))
    return ha, out.reshape(_B)


def kernel(x, indices, W1, b1, W2, b2):
    info = plsc.get_sparse_core_info()
    nw = info.num_cores * info.num_subcores
    th = _H // nw
    idx_flat = (
        indices.T.astype(jnp.int32).reshape(_K, nw, th).transpose(1, 0, 2).reshape(-1)
    )
    w1_flat = (
        W1.T.astype(jnp.float32).reshape(_K, nw, th).transpose(1, 0, 2).reshape(-1)
    )
    mt = _build_mt_sparsecore(idx_flat, w1_flat).reshape(_H, _D)
    return _forward_tensorcore(x, mt, b1, W2, b2)


# speedup vs baseline: 1.0032x; 1.0032x over previous
import functools

import jax
import jax.numpy as jnp
from jax import lax
from jax.experimental import pallas as pl
from jax.experimental.pallas import tpu as pltpu
from jax.experimental.pallas import tpu_sc as plsc

_B, _D, _H, _K = 4096, 512, 512, 16
_LANES = 16


def _build_mt_sparsecore(idx_flat, w1_flat):
    info = plsc.get_sparse_core_info()
    nw = info.num_cores * info.num_subcores
    th = _H // nw
    blk = _K * th
    rowlen = th * _D

    mesh = plsc.VectorSubcoreMesh(core_axis_name="c", subcore_axis_name="s")

    @functools.partial(
        pl.kernel,
        mesh=mesh,
        compiler_params=pltpu.CompilerParams(needs_layout_passes=False),
        out_type=jax.ShapeDtypeStruct((_H * _D,), jnp.float32),
        scratch_types=[
            pltpu.VMEM((blk,), jnp.int32),
            pltpu.VMEM((blk,), jnp.float32),
            pltpu.VMEM((rowlen,), jnp.float32),
        ],
    )
    def build(idx_hbm, w_hbm, m_hbm, idx_v, w_v, m_v):
        wid = lax.axis_index("s") * info.num_cores + lax.axis_index("c")
        pltpu.sync_copy(idx_hbm.at[pl.ds(wid * blk, blk)], idx_v)
        pltpu.sync_copy(w_hbm.at[pl.ds(wid * blk, blk)], w_v)

        def zero_chunk(i, c):
            m_v[pl.ds(i * _LANES, _LANES)] = jnp.zeros((_LANES,), jnp.float32)
            return c

        lax.fori_loop(0, rowlen // _LANES, zero_chunk, 0)

        lane = lax.broadcasted_iota(jnp.int32, (_LANES,), 0)
        row_off = lane * _D
        tap0 = lane * _K
        for k in range(_K):
            taps = plsc.load_gather(idx_v, [tap0 + k])
            wk = plsc.load_gather(w_v, [tap0 + k])
            plsc.addupdate_scatter(m_v, [row_off + taps], wk)

        pltpu.sync_copy(m_v, m_hbm.at[pl.ds(wid * rowlen, rowlen)])

    return build(idx_flat, w1_flat)


def _forward_body(x_ref, mt_ref, b1_ref, w2_ref, b2_ref, ha_ref, out_ref):
    dims = (((1,), (1,)), ((), ()))
    xf = x_ref[...]
    mf = mt_ref[...]
    xh = xf.astype(jnp.bfloat16)
    mh = mf.astype(jnp.bfloat16)
    xl = (xf - xh.astype(jnp.float32)).astype(jnp.bfloat16)
    ml = (mf - mh.astype(jnp.float32)).astype(jnp.bfloat16)
    h = lax.dot_general(
        xh, mh, dimension_numbers=dims, preferred_element_type=jnp.float32
    )
    h += lax.dot_general(
        xh, ml, dimension_numbers=dims, preferred_element_type=jnp.float32
    )
    h += lax.dot_general(
        xl, mh, dimension_numbers=dims, preferred_element_type=jnp.float32
    )
    ha = jnp.tanh(h + b1_ref[...])
    ha_ref[...] = ha
    o = jnp.sum(ha * w2_ref[...], axis=1, keepdims=True)
    out_ref[...] = jnp.tanh(o + b2_ref[...])


def _forward_tensorcore(x, mt, b1, w2, b2):
    bb = 512
    grid = (_B // bb,)
    ha, out = pl.pallas_call(
        _forward_body,
        grid=grid,
        in_specs=[
            pl.BlockSpec((bb, _D), lambda i: (i, 0)),
            pl.BlockSpec((_H, _D), lambda i: (0, 0)),
            pl.BlockSpec((1, _H), lambda i: (0, 0)),
            pl.BlockSpec((1, _H), lambda i: (0, 0)),
            pl.BlockSpec((1, 1), lambda i: (0, 0)),
        ],
        out_specs=[
            pl.BlockSpec((bb, _H), lambda i: (i, 0)),
            pl.BlockSpec((bb, 1), lambda i: (i, 0)),
        ],
        out_shape=[
            jax.ShapeDtypeStruct((_B, _H), jnp.float32),
            jax.ShapeDtypeStruct((_B, 1), jnp.float32),
        ],
    )(x, mt, b1.reshape(1, _H), w2.reshape(1, _H), b2.reshape(1, 1))
    return ha, out.reshape(_B)


def kernel(x, indices, W1, b1, W2, b2):
    idx_flat = indices.astype(jnp.int32).reshape(-1)
    w1_flat = W1.astype(jnp.float32).reshape(-1)
    mt = _build_mt_sparsecore(idx_flat, w1_flat).reshape(_H, _D)
    return _forward_tensorcore(x, mt, b1, W2, b2)


# speedup vs baseline: 1.0498x; 1.0464x over previous
import functools

import jax
import jax.numpy as jnp
from jax import lax
from jax.experimental import pallas as pl
from jax.experimental.pallas import tpu as pltpu
from jax.experimental.pallas import tpu_sc as plsc

_B, _D, _H, _K = 4096, 512, 512, 16
_LANES = 16


def _build_mt_sparsecore(indices, w1):
    info = plsc.get_sparse_core_info()
    nw = info.num_cores * info.num_subcores
    th = _H // nw

    mesh = plsc.VectorSubcoreMesh(core_axis_name="c", subcore_axis_name="s")

    @functools.partial(
        pl.kernel,
        mesh=mesh,
        compiler_params=pltpu.CompilerParams(needs_layout_passes=False),
        out_type=jax.ShapeDtypeStruct((_H, _D), jnp.float32),
        scratch_types=[
            pltpu.VMEM((th, _K), jnp.int32),
            pltpu.VMEM((th, _K), jnp.float32),
            pltpu.VMEM((th, _D), jnp.float32),
        ],
    )
    def build(idx_hbm, w_hbm, m_hbm, idx_v, w_v, m_v):
        wid = lax.axis_index("s") * info.num_cores + lax.axis_index("c")
        base = wid * th
        pltpu.sync_copy(idx_hbm.at[pl.ds(base, th), :], idx_v)
        pltpu.sync_copy(w_hbm.at[pl.ds(base, th), :], w_v)

        def zero_chunk(i, c):
            m_v[i >> 5, pl.ds((i & 31) * _LANES, _LANES)] = jnp.zeros(
                (_LANES,), jnp.float32
            )
            return c

        lax.fori_loop(0, th * (_D // _LANES), zero_chunk, 0)

        lane = lax.broadcasted_iota(jnp.int32, (_LANES,), 0)
        for k in range(_K):
            kvec = jnp.full((_LANES,), k, jnp.int32)
            taps = plsc.load_gather(idx_v, [lane, kvec])
            wk = plsc.load_gather(w_v, [lane, kvec])
            plsc.addupdate_scatter(m_v, [lane, taps], wk)

        pltpu.sync_copy(m_v, m_hbm.at[pl.ds(base, th), :])

    return build(indices, w1)


def _forward_body(x_ref, mt_ref, b1_ref, w2_ref, b2_ref, ha_ref, out_ref):
    dims = (((1,), (1,)), ((), ()))
    xf = x_ref[...]
    mf = mt_ref[...]
    xh = xf.astype(jnp.bfloat16)
    mh = mf.astype(jnp.bfloat16)
    xl = (xf - xh.astype(jnp.float32)).astype(jnp.bfloat16)
    ml = (mf - mh.astype(jnp.float32)).astype(jnp.bfloat16)
    h = lax.dot_general(
        xh, mh, dimension_numbers=dims, preferred_element_type=jnp.float32
    )
    h += lax.dot_general(
        xh, ml, dimension_numbers=dims, preferred_element_type=jnp.float32
    )
    h += lax.dot_general(
        xl, mh, dimension_numbers=dims, preferred_element_type=jnp.float32
    )
    ha = jnp.tanh(h + b1_ref[...])
    ha_ref[...] = ha
    o = jnp.sum(ha * w2_ref[...], axis=1, keepdims=True)
    out_ref[...] = jnp.tanh(o + b2_ref[...])


def _forward_tensorcore(x, mt, b1, w2, b2):
    bb = 512
    grid = (_B // bb,)
    ha, out = pl.pallas_call(
        _forward_body,
        grid=grid,
        in_specs=[
            pl.BlockSpec((bb, _D), lambda i: (i, 0)),
            pl.BlockSpec((_H, _D), lambda i: (0, 0)),
            pl.BlockSpec((1, _H), lambda i: (0, 0)),
            pl.BlockSpec((1, _H), lambda i: (0, 0)),
            pl.BlockSpec((1, 1), lambda i: (0, 0)),
        ],
        out_specs=[
            pl.BlockSpec((bb, _H), lambda i: (i, 0)),
            pl.BlockSpec((bb, 1), lambda i: (i, 0)),
        ],
        out_shape=[
            jax.ShapeDtypeStruct((_B, _H), jnp.float32),
            jax.ShapeDtypeStruct((_B, 1), jnp.float32),
        ],
    )(x, mt, b1.reshape(1, _H), w2.reshape(1, _H), b2.reshape(1, 1))
    return ha, out.reshape(_B)


def kernel(x, indices, W1, b1, W2, b2):
    mt = _build_mt_sparsecore(indices.astype(jnp.int32), W1.astype(jnp.float32))
    return _forward_tensorcore(x, mt, b1, W2, b2)


# speedup vs baseline: 1.1332x; 1.0795x over previous
import functools

import jax
import jax.numpy as jnp
from jax import lax
from jax.experimental import pallas as pl
from jax.experimental.pallas import tpu as pltpu
from jax.experimental.pallas import tpu_sc as plsc

_B, _D, _H, _K = 4096, 512, 512, 16
_LANES = 16


def _build_mt_sparsecore(indices, w1):
    info = plsc.get_sparse_core_info()
    nw = info.num_cores * info.num_subcores
    th = _H // nw

    mesh = plsc.VectorSubcoreMesh(core_axis_name="c", subcore_axis_name="s")

    @functools.partial(
        pl.kernel,
        mesh=mesh,
        compiler_params=pltpu.CompilerParams(needs_layout_passes=False),
        out_type=jax.ShapeDtypeStruct((_H, _D), jnp.float32),
        scratch_types=[
            pltpu.VMEM((th, _K), jnp.int32),
            pltpu.VMEM((th, _K), jnp.float32),
            pltpu.VMEM((th, _D), jnp.float32),
        ],
    )
    def build(idx_hbm, w_hbm, zero_hbm, m_hbm, idx_v, w_v, m_v):
        wid = lax.axis_index("s") * info.num_cores + lax.axis_index("c")
        base = wid * th
        pltpu.sync_copy(idx_hbm.at[pl.ds(base, th), :], idx_v)
        pltpu.sync_copy(w_hbm.at[pl.ds(base, th), :], w_v)
        pltpu.sync_copy(zero_hbm, m_v)

        lane = lax.broadcasted_iota(jnp.int32, (_LANES,), 0)

        def scatter_k(k, c):
            kvec = lane * 0 + k
            taps = plsc.load_gather(idx_v, [lane, kvec])
            wk = plsc.load_gather(w_v, [lane, kvec])
            plsc.addupdate_scatter(m_v, [lane, taps], wk)
            return c

        lax.fori_loop(0, _K, scatter_k, 0)

        pltpu.sync_copy(m_v, m_hbm.at[pl.ds(base, th), :])

    zeros = jnp.zeros((th, _D), jnp.float32)
    return build(indices, w1, zeros)


def _forward_body(x_ref, mt_ref, b1_ref, w2_ref, b2_ref, ha_ref, out_ref):
    dims = (((1,), (1,)), ((), ()))
    xf = x_ref[...]
    mf = mt_ref[...]
    xh = xf.astype(jnp.bfloat16)
    mh = mf.astype(jnp.bfloat16)
    xl = (xf - xh.astype(jnp.float32)).astype(jnp.bfloat16)
    ml = (mf - mh.astype(jnp.float32)).astype(jnp.bfloat16)
    h = lax.dot_general(
        xh, mh, dimension_numbers=dims, preferred_element_type=jnp.float32
    )
    h += lax.dot_general(
        xh, ml, dimension_numbers=dims, preferred_element_type=jnp.float32
    )
    h += lax.dot_general(
        xl, mh, dimension_numbers=dims, preferred_element_type=jnp.float32
    )
    ha = jnp.tanh(h + b1_ref[...])
    ha_ref[...] = ha
    o = jnp.sum(ha * w2_ref[...], axis=1, keepdims=True)
    out_ref[...] = jnp.tanh(o + b2_ref[...]).T


def _forward_tensorcore(x, mt, b1, w2, b2):
    bb = 1024
    grid = (_B // bb,)
    ha, out = pl.pallas_call(
        _forward_body,
        grid=grid,
        in_specs=[
            pl.BlockSpec((bb, _D), lambda i: (i, 0)),
            pl.BlockSpec((_H, _D), lambda i: (0, 0)),
            pl.BlockSpec((1, _H), lambda i: (0, 0)),
            pl.BlockSpec((1, _H), lambda i: (0, 0)),
            pl.BlockSpec((1, 1), lambda i: (0, 0)),
        ],
        out_specs=[
            pl.BlockSpec((bb, _H), lambda i: (i, 0)),
            pl.BlockSpec((1, bb), lambda i: (0, i)),
        ],
        out_shape=[
            jax.ShapeDtypeStruct((_B, _H), jnp.float32),
            jax.ShapeDtypeStruct((1, _B), jnp.float32),
        ],
    )(x, mt, b1.reshape(1, _H), w2.reshape(1, _H), b2.reshape(1, 1))
    return ha, out.reshape(_B)


def kernel(x, indices, W1, b1, W2, b2):
    mt = _build_mt_sparsecore(indices.astype(jnp.int32), W1.astype(jnp.float32))
    return _forward_tensorcore(x, mt, b1, W2, b2)


# speedup vs baseline: 1.2393x; 1.0937x over previous
import functools

import jax
import jax.numpy as jnp
from jax import lax
from jax.experimental import pallas as pl
from jax.experimental.pallas import tpu as pltpu
from jax.experimental.pallas import tpu_sc as plsc

_B, _D, _H, _K = 4096, 512, 512, 16
_LANES = 16


def _build_mt_sparsecore(idx2, w12):
    info = plsc.get_sparse_core_info()
    nw = info.num_cores * info.num_subcores
    th = _H // nw

    mesh = plsc.VectorSubcoreMesh(core_axis_name="c", subcore_axis_name="s")

    @functools.partial(
        pl.kernel,
        mesh=mesh,
        compiler_params=pltpu.CompilerParams(needs_layout_passes=False),
        out_type=jax.ShapeDtypeStruct((_H, _D), jnp.float32),
        scratch_types=[
            pltpu.VMEM((2, 128), jnp.int32),
            pltpu.VMEM((2, 128), jnp.float32),
            pltpu.VMEM((th, _D), jnp.float32),
        ],
    )
    def build(idx_hbm, w_hbm, m_hbm, idx_v, w_v, m_v):
        wid = lax.axis_index("s") * info.num_cores + lax.axis_index("c")
        base = wid * th
        pltpu.sync_copy(idx_hbm.at[pl.ds(wid * 2, 2), :], idx_v)
        pltpu.sync_copy(w_hbm.at[pl.ds(wid * 2, 2), :], w_v)

        zero = jnp.zeros((_LANES,), jnp.float32)

        def zero_chunk(i, c):
            for j in range(th):
                m_v[j, pl.ds(i * _LANES, _LANES)] = zero
            return c

        lax.fori_loop(0, _D // _LANES, zero_chunk, 0)

        lane = lax.broadcasted_iota(jnp.int32, (_LANES,), 0)
        loc0 = lane * _K

        def scatter_k(k, c):
            loc = loc0 + k
            row = lax.shift_right_logical(loc, 7)
            col = lax.bitwise_and(loc, 127)
            taps = plsc.load_gather(idx_v, [row, col])
            wk = plsc.load_gather(w_v, [row, col])
            plsc.addupdate_scatter(m_v, [lane, taps], wk)
            return c

        lax.fori_loop(0, _K, scatter_k, 0)

        pltpu.sync_copy(m_v, m_hbm.at[pl.ds(base, th), :])

    return build(idx2, w12)


def _forward_body(x_ref, mt_ref, b1_ref, w2_ref, b2_ref, ha_ref, out_ref):
    dims = (((1,), (1,)), ((), ()))
    xf = x_ref[...]
    mf = mt_ref[...]
    xh = xf.astype(jnp.bfloat16)
    mh = mf.astype(jnp.bfloat16)
    xl = (xf - xh.astype(jnp.float32)).astype(jnp.bfloat16)
    ml = (mf - mh.astype(jnp.float32)).astype(jnp.bfloat16)
    h = lax.dot_general(
        xh, mh, dimension_numbers=dims, preferred_element_type=jnp.float32
    )
    h += lax.dot_general(
        xh, ml, dimension_numbers=dims, preferred_element_type=jnp.float32
    )
    h += lax.dot_general(
        xl, mh, dimension_numbers=dims, preferred_element_type=jnp.float32
    )
    ha = jnp.tanh(h + b1_ref[...])
    ha_ref[...] = ha
    o = jnp.sum(ha * w2_ref[...], axis=1, keepdims=True)
    out_ref[...] = jnp.tanh(o + b2_ref[...]).T


def _forward_tensorcore(x, mt, b1, w2, b2):
    bb = 2048
    grid = (_B // bb,)
    ha, out = pl.pallas_call(
        _forward_body,
        grid=grid,
        in_specs=[
            pl.BlockSpec((bb, _D), lambda i: (i, 0)),
            pl.BlockSpec((_H, _D), lambda i: (0, 0)),
            pl.BlockSpec((1, _H), lambda i: (0, 0)),
            pl.BlockSpec((1, _H), lambda i: (0, 0)),
            pl.BlockSpec((1, 1), lambda i: (0, 0)),
        ],
        out_specs=[
            pl.BlockSpec((bb, _H), lambda i: (i, 0)),
            pl.BlockSpec((1, bb), lambda i: (0, i)),
        ],
        out_shape=[
            jax.ShapeDtypeStruct((_B, _H), jnp.float32),
            jax.ShapeDtypeStruct((1, _B), jnp.float32),
        ],
    )(x, mt, b1.reshape(1, _H), w2.reshape(1, _H), b2.reshape(1, 1))
    return ha, out.reshape(_B)


def kernel(x, indices, W1, b1, W2, b2):
    idx2 = indices.astype(jnp.int32).reshape(_H * _K // 128, 128)
    w12 = W1.astype(jnp.float32).reshape(_H * _K // 128, 128)
    mt = _build_mt_sparsecore(idx2, w12)
    return _forward_tensorcore(x, mt, b1, W2, b2)


# speedup vs baseline: 1.2709x; 1.0255x over previous
import functools

import jax
import jax.numpy as jnp
from jax import lax
from jax.experimental import pallas as pl
from jax.experimental.pallas import tpu as pltpu
from jax.experimental.pallas import tpu_sc as plsc

_B, _D, _H, _K = 4096, 512, 512, 16
_LANES = 16


def _build_mt_sparsecore(idx2, w12):
    info = plsc.get_sparse_core_info()
    nw = info.num_cores * info.num_subcores
    th = _H // nw

    mesh = plsc.VectorSubcoreMesh(core_axis_name="c", subcore_axis_name="s")

    @functools.partial(
        pl.kernel,
        mesh=mesh,
        compiler_params=pltpu.CompilerParams(needs_layout_passes=False),
        out_type=jax.ShapeDtypeStruct((_H, _D), jnp.float32),
        scratch_types=[
            pltpu.VMEM((2, 128), jnp.int32),
            pltpu.VMEM((2, 128), jnp.float32),
            pltpu.VMEM((th, _D), jnp.float32),
        ],
    )
    def build(idx_hbm, w_hbm, m_hbm, idx_v, w_v, m_v):
        wid = lax.axis_index("s") * info.num_cores + lax.axis_index("c")
        base = wid * th
        pltpu.sync_copy(idx_hbm.at[pl.ds(wid * 2, 2), :], idx_v)
        pltpu.sync_copy(w_hbm.at[pl.ds(wid * 2, 2), :], w_v)

        zero = jnp.zeros((_LANES,), jnp.float32)

        def zero_chunk(i, c):
            for j in range(th):
                m_v[j, pl.ds(i * _LANES, _LANES)] = zero
            return c

        lax.fori_loop(0, _D // _LANES, zero_chunk, 0)

        lane = lax.broadcasted_iota(jnp.int32, (_LANES,), 0)
        loc0 = lane * _K

        def scatter_k(k, c):
            loc = loc0 + k
            row = lax.shift_right_logical(loc, 7)
            col = lax.bitwise_and(loc, 127)
            taps = plsc.load_gather(idx_v, [row, col])
            wk = plsc.load_gather(w_v, [row, col])
            plsc.addupdate_scatter(m_v, [lane, taps], wk)
            return c

        lax.fori_loop(0, _K, scatter_k, 0)

        pltpu.sync_copy(m_v, m_hbm.at[pl.ds(base, th), :])

    return build(idx2, w12)


def _forward_body(x_ref, mt_ref, b1_ref, w2_ref, b2_ref, ha_ref, out_ref):
    dims = (((1,), (1,)), ((), ()))
    xf = x_ref[...]
    mf = mt_ref[...]
    xh = xf.astype(jnp.bfloat16)
    mh = mf.astype(jnp.bfloat16)
    xl = (xf - xh.astype(jnp.float32)).astype(jnp.bfloat16)
    ml = (mf - mh.astype(jnp.float32)).astype(jnp.bfloat16)
    h = lax.dot_general(
        xh, mh, dimension_numbers=dims, preferred_element_type=jnp.float32
    )
    h += lax.dot_general(
        xh, ml, dimension_numbers=dims, preferred_element_type=jnp.float32
    )
    h += lax.dot_general(
        xl, mh, dimension_numbers=dims, preferred_element_type=jnp.float32
    )
    ha = jnp.tanh(h + b1_ref[...])
    ha_ref[...] = ha
    o = jnp.sum(ha * w2_ref[...], axis=1, keepdims=True)
    out_ref[...] = jnp.tanh(o + b2_ref[...]).T


def _forward_tensorcore(x, mt, b1, w2, b2):
    bb = 1024
    grid = (_B // bb,)
    ha, out = pl.pallas_call(
        _forward_body,
        grid=grid,
        compiler_params=pltpu.CompilerParams(
            dimension_semantics=("parallel",)
        ),
        in_specs=[
            pl.BlockSpec((bb, _D), lambda i: (i, 0)),
            pl.BlockSpec((_H, _D), lambda i: (0, 0)),
            pl.BlockSpec((1, _H), lambda i: (0, 0)),
            pl.BlockSpec((1, _H), lambda i: (0, 0)),
            pl.BlockSpec((1, 1), lambda i: (0, 0)),
        ],
        out_specs=[
            pl.BlockSpec((bb, _H), lambda i: (i, 0)),
            pl.BlockSpec((1, bb), lambda i: (0, i)),
        ],
        out_shape=[
            jax.ShapeDtypeStruct((_B, _H), jnp.float32),
            jax.ShapeDtypeStruct((1, _B), jnp.float32),
        ],
    )(x, mt, b1.reshape(1, _H), w2.reshape(1, _H), b2.reshape(1, 1))
    return ha, out.reshape(_B)


def kernel(x, indices, W1, b1, W2, b2):
    idx2 = indices.reshape(_H * _K // 128, 128)
    w12 = W1.reshape(_H * _K // 128, 128)
    mt = _build_mt_sparsecore(idx2, w12)
    return _forward_tensorcore(x, mt, b1, W2, b2)
